# baseline (speedup 1.0000x reference)
import functools

import jax
import jax.numpy as jnp
from jax import lax
from jax.experimental import pallas as pl
from jax.experimental.pallas import tpu as pltpu
from jax.experimental.pallas import tpu_sc as plsc

DIM = 64
CHUNK = 128
NBUF = 8


def _build_gather(num_chunks_total: int):
    info = plsc.get_sparse_core_info()
    nc, ns = info.num_cores, info.num_subcores
    nw = nc * ns
    chunks_per_w = num_chunks_total // nw
    groups = chunks_per_w // NBUF
    b = num_chunks_total * CHUNK

    mesh = plsc.VectorSubcoreMesh(core_axis_name="c", subcore_axis_name="s")

    scratch = [pltpu.VMEM((chunks_per_w, CHUNK), jnp.int32)]
    scratch += [pltpu.VMEM((CHUNK, DIM), jnp.float32) for _ in range(NBUF)]
    scratch += [pltpu.SemaphoreType.DMA, pltpu.SemaphoreType.DMA]

    @functools.partial(
        pl.kernel,
        mesh=mesh,
        compiler_params=pltpu.CompilerParams(use_tc_tiling_on_sc=False),
        out_type=jax.ShapeDtypeStruct((b, DIM), jnp.float32),
        scratch_types=scratch,
    )
    def gather_kernel(table_hbm, idx_hbm, out_hbm, idx_v, *rest):
        bufs = rest[:NBUF]
        gsem, ssem = rest[NBUF], rest[NBUF + 1]
        wid = lax.axis_index("s") * nc + lax.axis_index("c")
        cbase = wid * chunks_per_w
        pltpu.sync_copy(idx_hbm.at[pl.ds(cbase, chunks_per_w)], idx_v)

        def group_body(g, carry):
            j0 = g * NBUF
            gets = [
                pltpu.async_copy(table_hbm.at[idx_v.at[j0 + k]], bufs[k], gsem)
                for k in range(NBUF)
            ]
            for d in gets:
                d.wait()
            puts = [
                pltpu.async_copy(
                    bufs[k],
                    out_hbm.at[pl.ds((cbase + j0 + k) * CHUNK, CHUNK)],
                    ssem,
                )
                for k in range(NBUF)
            ]
            for d in puts:
                d.wait()
            return carry

        lax.fori_loop(0, groups, group_body, 0)

    return gather_kernel


def kernel(level_idx, embedding_table):
    orig_shape = level_idx.shape
    idx = level_idx.reshape(-1).astype(jnp.int32)
    num_chunks = idx.shape[0] // CHUNK
    idx2d = idx.reshape(num_chunks, CHUNK)
    out = _build_gather(num_chunks)(embedding_table, idx2d)
    return out.reshape(orig_shape + (DIM,))
